# baseline (speedup 1.0000x reference)
import jax
import jax.numpy as jnp
from jax import lax
from jax.experimental import pallas as pl
from jax.experimental.pallas import tpu as pltpu
from jax.experimental.pallas import tpu_sc as plsc

SIZE = 1000000
DIM = 64
BATCH = 16384
BAG = 50
BAGP = 64

NCORE = 2
NSUB = 16
NW = NCORE * NSUB
BPT = BATCH // NW
CB = 4
NCH = BPT // CB
SUB = 2
IPS = CB * BAG // SUB
LANES = 16
DV = DIM // LANES
LINE = 2 * DIM


def _bag_body(idx_hbm, scr_hbm, tbl_hbm, out_hbm, idx_v, scr_v, rows_v, out_v,
              sem0, sem1):
    wid = lax.axis_index("s") * NCORE + lax.axis_index("c")
    sems = (sem0, sem1)

    pltpu.sync_copy(idx_hbm.at[wid], idx_v)
    pltpu.sync_copy(scr_hbm.at[wid], scr_v)

    def issue(g, b):
        for s in range(SUB):
            pltpu.async_copy(
                tbl_hbm.at[idx_v.at[g * SUB + s]],
                rows_v.at[b, pl.ds(s * IPS, IPS)],
                sems[b],
            )

    def drain(g, b):
        for s in range(SUB):
            pltpu.make_async_copy(
                tbl_hbm.at[idx_v.at[g * SUB + s]],
                rows_v.at[b, pl.ds(s * IPS, IPS)],
                sems[b],
            ).wait()

    def compute(g, b, half):
        def bag(c, _):
            accs = [jnp.zeros((LANES,), jnp.float32) for _ in range(DV)]
            cg = g * CB + c
            srow = lax.shift_right_logical(cg, 1)
            scol = lax.mul(lax.bitwise_and(cg, 1), BAGP)
            for jj in range(0, BAG, LANES):
                svec = scr_v[srow, pl.ds(scol + jj, LANES)]
                for lane in range(min(LANES, BAG - jj)):
                    j = jj + lane
                    raw = svec[lane]
                    sc = lax.abs(raw) - 1.0
                    off = lax.select(raw < 0.0, DIM, 0)
                    r = c * BAG + j
                    for t in range(DV):
                        accs[t] = accs[t] + sc * rows_v[
                            b, r, pl.ds(off + t * LANES, LANES)]
            for t in range(DV):
                out_v[half * CB + c, pl.ds(t * LANES, LANES)] = accs[t]
            return 0

        lax.fori_loop(0, CB, bag, 0)

    issue(0, 0)

    def pair(gg, _):
        g0 = 2 * gg
        g1 = g0 + 1
        issue(g1, 1)
        drain(g0, 0)
        compute(g0, 0, 0)

        @pl.when(g1 + 1 < NCH)
        def _():
            issue(g1 + 1, 0)

        drain(g1, 1)
        compute(g1, 1, 1)
        pltpu.sync_copy(out_v, out_hbm.at[pl.ds(wid * BPT + gg * 2 * CB,
                                                2 * CB)])
        return 0

    lax.fori_loop(0, NCH // 2, pair, 0)


@jax.jit
def _bag_call(idx3, scr3, tbl2):
    mesh = plsc.VectorSubcoreMesh(core_axis_name="c", subcore_axis_name="s")
    return pl.kernel(
        _bag_body,
        out_type=jax.ShapeDtypeStruct((BATCH, DIM), jnp.float32),
        mesh=mesh,
        scratch_types=[
            pltpu.VMEM((NCH * SUB, IPS), jnp.int32),
            pltpu.VMEM((BPT * BAGP // 128, 128), jnp.float32),
            pltpu.VMEM((2, CB * BAG, LINE), jnp.float32),
            pltpu.VMEM((2 * CB, DIM), jnp.float32),
            pltpu.SemaphoreType.DMA,
            pltpu.SemaphoreType.DMA,
        ],
        compiler_params=pltpu.CompilerParams(use_tc_tiling_on_sc=True),
    )(idx3, scr3, tbl2)


def kernel(indices, scores, weight):
    idx = indices.astype(jnp.int32)
    idx3 = lax.shift_right_logical(idx, 1).reshape(NW, NCH * SUB, IPS)
    parity = jnp.float32(1.0) - 2.0 * (idx & 1).astype(jnp.float32)
    spk = (scores + 1.0) * parity
    spk = jnp.pad(spk, ((0, 0), (0, BAGP - BAG)))
    scr3 = spk.reshape(NW, BPT * BAGP // 128, 128)
    tbl2 = weight.reshape(SIZE // 2, LINE)
    return _bag_call(idx3, scr3, tbl2)
